# Initial kernel scaffold; baseline (speedup 1.0000x reference)
import jax
import jax.numpy as jnp
from jax.experimental import pallas as pl


def kernel(inputs, table, W, b):
    raise NotImplementedError("write your pallas kernel here")


# speedup vs baseline: 2.4868x; 2.4868x over previous
import functools

import jax
import jax.numpy as jnp
from jax import lax
from jax.experimental import pallas as pl
from jax.experimental.pallas import tpu as pltpu
from jax.experimental.pallas import tpu_sc as plsc

VOCAB = 1000000
EMBED_DIM = 32
FOLD = 8
ROW_BLOCK = 5000

NUM_CORES = 2
NUM_SUBCORES = 16
NUM_WORKERS = NUM_CORES * NUM_SUBCORES


def _tv_body(tab_ref, wm_ref, b_ref, out_ref):
    acc = jnp.dot(tab_ref[...], wm_ref[...], preferred_element_type=jnp.float32)
    out_ref[...] = jax.nn.sigmoid(acc + b_ref[0, 0])


def _compute_tv(table, W, b):
    v_fold = VOCAB // FOLD
    k_dim = FOLD * EMBED_DIM
    tab = table.reshape(v_fold, k_dim)
    wvec = W.reshape(EMBED_DIM)
    wm = (jnp.eye(FOLD, dtype=jnp.float32)[:, None, :]
          * wvec[None, :, None]).reshape(k_dim, FOLD)
    b2 = b.reshape(1, 1)
    grid = v_fold // ROW_BLOCK
    out = pl.pallas_call(
        _tv_body,
        grid=(grid,),
        in_specs=[
            pl.BlockSpec((ROW_BLOCK, k_dim), lambda i: (i, 0)),
            pl.BlockSpec((k_dim, FOLD), lambda i: (0, 0)),
            pl.BlockSpec(memory_space=pltpu.SMEM),
        ],
        out_specs=pl.BlockSpec((ROW_BLOCK, FOLD), lambda i: (i, 0)),
        out_shape=jax.ShapeDtypeStruct((v_fold, FOLD), jnp.float32),
    )(tab, wm, b2)
    return out.reshape(VOCAB)


def _make_gather(total):
    chunk = total // NUM_WORKERS
    mesh = plsc.VectorSubcoreMesh(core_axis_name="c", subcore_axis_name="s")

    @functools.partial(
        pl.kernel,
        mesh=mesh,
        out_type=jax.ShapeDtypeStruct((total,), jnp.float32),
        scratch_types=[
            pltpu.VMEM((chunk,), jnp.int32),
            pltpu.VMEM((chunk,), jnp.float32),
            pltpu.SemaphoreType.DMA,
        ],
    )
    def gather(tv_hbm, idx_hbm, out_hbm, idx_v, val_v, sem):
        wid = lax.axis_index("s") * NUM_CORES + lax.axis_index("c")
        base = wid * chunk
        pltpu.sync_copy(idx_hbm.at[pl.ds(base, chunk)], idx_v)
        pltpu.async_copy(tv_hbm.at[idx_v], val_v, sem).wait()
        pltpu.sync_copy(val_v, out_hbm.at[pl.ds(base, chunk)])

    return gather


def kernel(inputs, table, W, b):
    batch, hist = inputs.shape
    total = batch * hist
    idx = inputs.reshape(total).astype(jnp.int32)
    tv = _compute_tv(table, W, b)
    out = _make_gather(total)(tv, idx)
    return out.reshape(batch, hist, 1)


# speedup vs baseline: 2.6812x; 1.0782x over previous
import functools

import jax
import jax.numpy as jnp
from jax import lax
from jax.experimental import pallas as pl
from jax.experimental.pallas import tpu as pltpu
from jax.experimental.pallas import tpu_sc as plsc

VOCAB = 1000000
EMBED_DIM = 32
FOLD = 8
ROW_BLOCK = 4096
TV_GRID = 31
G_PER_STEP = ROW_BLOCK // 128
TV_PAD = TV_GRID * ROW_BLOCK * FOLD

NUM_CORES = 2
NUM_SUBCORES = 16
NUM_WORKERS = NUM_CORES * NUM_SUBCORES
LANES = 16


def _tv_body(tab_ref, wmt_ref, b_ref, out_ref):
    acc_t = lax.dot_general(
        wmt_ref[...], tab_ref[...],
        dimension_numbers=(((1,), (1,)), ((), ())),
        preferred_element_type=jnp.float32,
    )
    y = jax.nn.sigmoid(acc_t + b_ref[0, 0])
    for g in range(G_PER_STEP):
        out_ref[g] = y[:, g * 128:(g + 1) * 128]


def _compute_tv(table, W, b):
    v_fold = VOCAB // FOLD
    k_dim = FOLD * EMBED_DIM
    tab = table.reshape(v_fold, k_dim)
    wvec = W.reshape(EMBED_DIM)
    wmt = (jnp.eye(FOLD, dtype=jnp.float32)[:, :, None]
           * wvec[None, None, :]).reshape(FOLD, k_dim)
    b2 = b.reshape(1, 1)
    out = pl.pallas_call(
        _tv_body,
        grid=(TV_GRID,),
        in_specs=[
            pl.BlockSpec((ROW_BLOCK, k_dim), lambda i: (i, 0)),
            pl.BlockSpec((FOLD, k_dim), lambda i: (0, 0)),
            pl.BlockSpec(memory_space=pltpu.SMEM),
        ],
        out_specs=pl.BlockSpec((G_PER_STEP, FOLD, 128), lambda i: (i, 0, 0)),
        out_shape=jax.ShapeDtypeStruct((TV_GRID * G_PER_STEP, FOLD, 128),
                                       jnp.float32),
    )(tab, wmt, b2)
    return out.reshape(TV_PAD)


def _make_gather(total):
    chunk = total // NUM_WORKERS
    mesh = plsc.VectorSubcoreMesh(core_axis_name="c", subcore_axis_name="s")

    @functools.partial(
        pl.kernel,
        mesh=mesh,
        out_type=jax.ShapeDtypeStruct((total,), jnp.float32),
        scratch_types=[
            pltpu.VMEM((chunk,), jnp.int32),
            pltpu.VMEM((chunk,), jnp.float32),
            pltpu.SemaphoreType.DMA,
        ],
    )
    def gather(tv_hbm, idx_hbm, out_hbm, idx_v, val_v, sem):
        wid = lax.axis_index("s") * NUM_CORES + lax.axis_index("c")
        base = wid * chunk
        pltpu.sync_copy(idx_hbm.at[pl.ds(base, chunk)], idx_v)

        def remap(j, _):
            v = idx_v[pl.ds(j * LANES, LANES)]
            m = ((v & -1024) | ((v & 7) << 7) | ((v >> 3) & 127))
            idx_v[pl.ds(j * LANES, LANES)] = m
            return 0

        lax.fori_loop(0, chunk // LANES, remap, 0)
        pltpu.async_copy(tv_hbm.at[idx_v], val_v, sem).wait()
        pltpu.sync_copy(val_v, out_hbm.at[pl.ds(base, chunk)])

    return gather


def kernel(inputs, table, W, b):
    batch, hist = inputs.shape
    total = batch * hist
    idx = inputs.reshape(total).astype(jnp.int32)
    tv = _compute_tv(table, W, b)
    out = _make_gather(total)(tv, idx)
    return out.reshape(batch, hist, 1)


# speedup vs baseline: 11.6291x; 4.3373x over previous
import functools

import jax
import jax.numpy as jnp
from jax import lax
from jax.experimental import pallas as pl
from jax.experimental.pallas import tpu as pltpu
from jax.experimental.pallas import tpu_sc as plsc

VOCAB = 1000000
EMBED_DIM = 32
COL_BLOCK = 32768
TV_GRID = 31
TV_PAD = TV_GRID * COL_BLOCK

NUM_CORES = 2
NUM_SUBCORES = 16
NUM_WORKERS = NUM_CORES * NUM_SUBCORES


def _tv_body(tabt_ref, w_ref, b_ref, out_ref):
    acc = jnp.sum(tabt_ref[...] * w_ref[...], axis=0)
    out_ref[...] = jax.nn.sigmoid(acc + b_ref[0, 0])


def _compute_tv(table, W, b):
    tabt = table.T
    b2 = b.reshape(1, 1)
    out = pl.pallas_call(
        _tv_body,
        grid=(TV_GRID,),
        in_specs=[
            pl.BlockSpec((EMBED_DIM, COL_BLOCK), lambda i: (0, i)),
            pl.BlockSpec((EMBED_DIM, 1), lambda i: (0, 0)),
            pl.BlockSpec(memory_space=pltpu.SMEM),
        ],
        out_specs=pl.BlockSpec((COL_BLOCK,), lambda i: (i,)),
        out_shape=jax.ShapeDtypeStruct((TV_PAD,), jnp.float32),
    )(tabt, W, b2)
    return out


def _make_gather(total):
    chunk = total // NUM_WORKERS
    mesh = plsc.VectorSubcoreMesh(core_axis_name="c", subcore_axis_name="s")

    @functools.partial(
        pl.kernel,
        mesh=mesh,
        out_type=jax.ShapeDtypeStruct((total,), jnp.float32),
        scratch_types=[
            pltpu.VMEM((chunk,), jnp.int32),
            pltpu.VMEM((chunk,), jnp.float32),
            pltpu.SemaphoreType.DMA,
        ],
    )
    def gather(tv_hbm, idx_hbm, out_hbm, idx_v, val_v, sem):
        wid = lax.axis_index("s") * NUM_CORES + lax.axis_index("c")
        base = wid * chunk
        pltpu.sync_copy(idx_hbm.at[pl.ds(base, chunk)], idx_v)
        pltpu.async_copy(tv_hbm.at[idx_v], val_v, sem).wait()
        pltpu.sync_copy(val_v, out_hbm.at[pl.ds(base, chunk)])

    return gather


def kernel(inputs, table, W, b):
    batch, hist = inputs.shape
    total = batch * hist
    idx = inputs.reshape(total).astype(jnp.int32)
    tv = _compute_tv(table, W, b)
    out = _make_gather(total)(tv, idx)
    return out.reshape(batch, hist, 1)


# speedup vs baseline: 13.7559x; 1.1829x over previous
import functools

import jax
import jax.numpy as jnp
from jax import lax
from jax.experimental import pallas as pl
from jax.experimental.pallas import tpu as pltpu
from jax.experimental.pallas import tpu_sc as plsc

VOCAB = 1000000
EMBED_DIM = 32
COL_BLOCK = 32768
TV_GRID = 31
TV_PAD = TV_GRID * COL_BLOCK

NUM_CORES = 2
NUM_SUBCORES = 16
NUM_WORKERS = NUM_CORES * NUM_SUBCORES


def _tv_body(tabt_ref, w_ref, b_ref, out_ref):
    acc = jnp.sum(tabt_ref[...] * w_ref[...], axis=0)
    out_ref[...] = jax.nn.sigmoid(acc + b_ref[0, 0])


def _compute_tv(table, W, b):
    tabt = table.T
    b2 = b.reshape(1, 1)
    out = pl.pallas_call(
        _tv_body,
        grid=(TV_GRID,),
        in_specs=[
            pl.BlockSpec((EMBED_DIM, COL_BLOCK), lambda i: (0, i)),
            pl.BlockSpec((EMBED_DIM, 1), lambda i: (0, 0)),
            pl.BlockSpec(memory_space=pltpu.SMEM),
        ],
        out_specs=pl.BlockSpec((COL_BLOCK,), lambda i: (i,)),
        out_shape=jax.ShapeDtypeStruct((TV_PAD,), jnp.float32),
    )(tabt, W, b2)
    return out


def _make_gather(total):
    chunk = total // NUM_WORKERS
    mesh = plsc.VectorSubcoreMesh(core_axis_name="c", subcore_axis_name="s")

    @functools.partial(
        pl.kernel,
        mesh=mesh,
        out_type=jax.ShapeDtypeStruct((total,), jnp.float32),
        scratch_types=[
            pltpu.VMEM((chunk,), jnp.int32),
            pltpu.VMEM((chunk,), jnp.float32),
            pltpu.SemaphoreType.DMA,
        ],
    )
    def gather(tv_hbm, idx_hbm, out_hbm, idx_v, val_v, sem):
        wid = lax.axis_index("s") * NUM_CORES + lax.axis_index("c")
        base = wid * chunk
        pltpu.sync_copy(idx_hbm.at[pl.ds(base, chunk)], idx_v)
        pltpu.async_copy(tv_hbm.at[idx_v], val_v, sem).wait()
        pltpu.sync_copy(val_v, out_hbm.at[pl.ds(base, chunk)])

    return gather


def kernel(inputs, table, W, b):
    batch, hist = inputs.shape
    total = batch * hist
    idx = inputs.T.reshape(total).astype(jnp.int32)
    tv = _compute_tv(table, W, b)
    g = _make_gather(total)(tv, idx)
    return g.reshape(hist, batch).T.reshape(batch, hist, 1)


# speedup vs baseline: 16.5211x; 1.2010x over previous
import functools

import jax
import jax.numpy as jnp
from jax import lax
from jax.experimental import pallas as pl
from jax.experimental.pallas import tpu as pltpu
from jax.experimental.pallas import tpu_sc as plsc

VOCAB = 1000000
EMBED_DIM = 32
COL_BLOCK = 65536
TV_GRID = 16
TV_PAD = TV_GRID * COL_BLOCK

NUM_CORES = 2
NUM_SUBCORES = 16
NUM_WORKERS = NUM_CORES * NUM_SUBCORES


def _tv_body(tabt_ref, w_ref, b_ref, out_ref):
    acc = jnp.sum(tabt_ref[...] * w_ref[...], axis=0)
    out_ref[...] = jax.nn.sigmoid(acc + b_ref[0, 0])


def _compute_tv(table, W, b):
    tabt = table.T
    b2 = b.reshape(1, 1)
    out = pl.pallas_call(
        _tv_body,
        grid=(TV_GRID,),
        in_specs=[
            pl.BlockSpec((EMBED_DIM, COL_BLOCK), lambda i: (0, i)),
            pl.BlockSpec((EMBED_DIM, 1), lambda i: (0, 0)),
            pl.BlockSpec(memory_space=pltpu.SMEM),
        ],
        out_specs=pl.BlockSpec((COL_BLOCK,), lambda i: (i,)),
        out_shape=jax.ShapeDtypeStruct((TV_PAD,), jnp.float32),
    )(tabt, W, b2)
    return out


def _make_gather(total):
    chunk = total // NUM_WORKERS
    mesh = plsc.VectorSubcoreMesh(core_axis_name="c", subcore_axis_name="s")

    @functools.partial(
        pl.kernel,
        mesh=mesh,
        out_type=jax.ShapeDtypeStruct((total,), jnp.float32),
        scratch_types=[
            pltpu.VMEM((chunk,), jnp.int32),
            pltpu.VMEM((chunk,), jnp.float32),
            pltpu.SemaphoreType.DMA,
        ],
    )
    def gather(tv_hbm, idx_hbm, out_hbm, idx_v, val_v, sem):
        wid = lax.axis_index("s") * NUM_CORES + lax.axis_index("c")
        base = wid * chunk
        pltpu.sync_copy(idx_hbm.at[pl.ds(base, chunk)], idx_v)
        pltpu.async_copy(tv_hbm.at[idx_v], val_v, sem).wait()
        pltpu.sync_copy(val_v, out_hbm.at[pl.ds(base, chunk)])

    return gather


def kernel(inputs, table, W, b):
    batch, hist = inputs.shape
    total = batch * hist
    idx = inputs.T.reshape(total).astype(jnp.int32)
    tv = _compute_tv(table, W, b)
    g = _make_gather(total)(tv, idx)
    return g.reshape(hist, batch, 1).transpose(1, 0, 2)


# speedup vs baseline: 19.9030x; 1.2047x over previous
import functools

import jax
import jax.numpy as jnp
from jax import lax
from jax.experimental import pallas as pl
from jax.experimental.pallas import tpu as pltpu
from jax.experimental.pallas import tpu_sc as plsc

VOCAB = 1000000
EMBED_DIM = 32
COL_BLOCK = 81920
TV_GRID = 13
TV_PAD = TV_GRID * COL_BLOCK

NUM_CORES = 2
NUM_SUBCORES = 16
NUM_WORKERS = NUM_CORES * NUM_SUBCORES


def _tv_body(tabt_ref, w_ref, b_ref, out_ref):
    acc = jnp.sum(tabt_ref[...] * w_ref[...], axis=0)
    out_ref[...] = jax.nn.sigmoid(acc + b_ref[0, 0])


def _compute_tv(table, W, b):
    tabt = table.T
    b2 = b.reshape(1, 1)
    out = pl.pallas_call(
        _tv_body,
        grid=(TV_GRID,),
        in_specs=[
            pl.BlockSpec((EMBED_DIM, COL_BLOCK), lambda i: (0, i)),
            pl.BlockSpec((EMBED_DIM, 1), lambda i: (0, 0)),
            pl.BlockSpec(memory_space=pltpu.SMEM),
        ],
        out_specs=pl.BlockSpec((COL_BLOCK,), lambda i: (i,)),
        out_shape=jax.ShapeDtypeStruct((TV_PAD,), jnp.float32),
    )(tabt, W, b2)
    return out


def _make_gather(total):
    chunk = total // NUM_WORKERS
    tv_slice = TV_PAD // NUM_SUBCORES
    mesh = plsc.VectorSubcoreMesh(core_axis_name="c", subcore_axis_name="s")

    @functools.partial(
        pl.kernel,
        mesh=mesh,
        out_type=jax.ShapeDtypeStruct((total,), jnp.float32),
        scratch_types=[
            pltpu.VMEM((chunk,), jnp.int32),
            pltpu.VMEM((chunk,), jnp.float32),
            pltpu.VMEM_SHARED((TV_PAD,), jnp.float32),
            pltpu.SemaphoreType.DMA,
        ],
    )
    def gather(tv_hbm, idx_hbm, out_hbm, idx_v, val_v, tv_sh, sem):
        sid = lax.axis_index("s")
        wid = sid * NUM_CORES + lax.axis_index("c")
        base = wid * chunk
        pltpu.sync_copy(idx_hbm.at[pl.ds(base, chunk)], idx_v)
        off = sid * tv_slice
        pltpu.sync_copy(tv_hbm.at[pl.ds(off, tv_slice)],
                        tv_sh.at[pl.ds(off, tv_slice)])
        plsc.subcore_barrier()
        pltpu.async_copy(tv_sh.at[idx_v], val_v, sem).wait()
        pltpu.sync_copy(val_v, out_hbm.at[pl.ds(base, chunk)])

    return gather


def kernel(inputs, table, W, b):
    batch, hist = inputs.shape
    total = batch * hist
    idx = inputs.T.reshape(total).astype(jnp.int32)
    tv = _compute_tv(table, W, b)
    g = _make_gather(total)(tv, idx)
    return g.reshape(hist, batch, 1).transpose(1, 0, 2)
